# baseline (speedup 1.0000x reference)
import functools

import jax
import jax.numpy as jnp
from jax import lax
from jax.experimental import pallas as pl
from jax.experimental.pallas import tpu as pltpu
from jax.experimental.pallas import tpu_sc as plsc

_N = 10000
_E = 160000
_NC = 2
_NS = 16
_CH = 128
_CHUNK = 128
_NCHUNK = 1280
_CPT = _NCHUNK // _NS
_HALF = _CPT // 2
_NP = 10240
_RPT = _NP // _NS
_NB = 1000


def _sc_mesh():
    return plsc.VectorSubcoreMesh(core_axis_name="c", subcore_axis_name="s")


_NW = _NC * _NS
_CPW = _NCHUNK // _NW


def _sc_degree(dst2d, ones_hbm, zeros_hbm):
    @functools.partial(
        pl.kernel,
        out_type=jax.ShapeDtypeStruct((_NC, _NP, _CH), jnp.float32),
        mesh=_sc_mesh(),
        scratch_types=[
            pltpu.VMEM((_CPW, _CHUNK), jnp.int32),
            pltpu.VMEM((_CHUNK, _CH), jnp.float32),
            pltpu.VMEM_SHARED((_NP, _CH), jnp.float32),
        ],
    )
    def body(dst_hbm, ones_hbm_ref, zeros_hbm_ref, out_hbm, dstm, onesv, acc):
        cid = lax.axis_index("c")
        sid = lax.axis_index("s")
        wid = cid * _NS + sid

        pltpu.sync_copy(ones_hbm_ref, onesv)
        pltpu.sync_copy(zeros_hbm_ref.at[pl.ds(sid * _RPT, _RPT), :],
                        acc.at[pl.ds(sid * _RPT, _RPT), :])
        pltpu.sync_copy(dst_hbm.at[pl.ds(wid * _CPW, _CPW), :], dstm)
        plsc.subcore_barrier()

        def step(k, c):
            pltpu.sync_copy(onesv, acc.at[dstm.at[k]], add=True)
            return c

        lax.fori_loop(0, _CPW, step, 0)

        plsc.subcore_barrier()
        pltpu.sync_copy(acc.at[pl.ds(sid * _RPT, _RPT), :],
                        out_hbm.at[cid, pl.ds(sid * _RPT, _RPT), :])

    return body(dst2d, ones_hbm, zeros_hbm)


def _sc_aggregate(u_flat, src2d, dst2d):
    @functools.partial(
        pl.kernel,
        out_type=jax.ShapeDtypeStruct((_NC, _NP, _CH), jnp.float32),
        mesh=_sc_mesh(),
        scratch_types=[
            pltpu.VMEM((_HALF, _CHUNK), jnp.int32),
            pltpu.VMEM((_HALF, _CHUNK), jnp.int32),
            pltpu.VMEM((_CHUNK, _CH), jnp.float32),
            pltpu.VMEM((_CHUNK, _CH), jnp.float32),
            pltpu.VMEM_SHARED((_NP, _CH), jnp.float32),
            pltpu.SemaphoreType.DMA,
            pltpu.SemaphoreType.DMA,
        ],
    )
    def body(u_hbm, src_hbm, dst_hbm, out_hbm, srcm, dstm, rows0, rows1,
             acc, sem0, sem1):
        cid = lax.axis_index("c")
        sid = lax.axis_index("s")
        pltpu.sync_copy(u_hbm.at[pl.ds(cid * _NP + sid * _RPT, _RPT), :],
                        acc.at[pl.ds(sid * _RPT, _RPT), :])
        plsc.subcore_barrier()

        def gather(k, buf, sem):
            pltpu.async_copy(u_hbm.at[srcm.at[k]], buf, sem)

        def gwait(buf, sem):
            pltpu.make_async_copy(u_hbm.at[srcm.at[0]], buf, sem).wait()

        def scatter(k, buf):
            pltpu.sync_copy(buf, acc.at[dstm.at[k]], add=True)

        def step(j, c):
            a = 2 * j
            gather(a + 1, rows1, sem1)
            gwait(rows0, sem0)
            scatter(a, rows0)
            gather(jnp.minimum(a + 2, _HALF - 1), rows0, sem0)
            gwait(rows1, sem1)
            scatter(a + 1, rows1)
            return c

        for h in range(_CPT // _HALF):
            cbase = sid * _CPT + h * _HALF
            pltpu.sync_copy(src_hbm.at[pl.ds(cid * _NCHUNK + cbase, _HALF), :],
                            srcm)
            pltpu.sync_copy(dst_hbm.at[pl.ds(cbase, _HALF), :], dstm)
            gather(0, rows0, sem0)
            lax.fori_loop(0, _HALF // 2, step, 0)
            gwait(rows0, sem0)

        plsc.subcore_barrier()
        pltpu.sync_copy(acc.at[pl.ds(sid * _RPT, _RPT), :],
                        out_hbm.at[cid, pl.ds(sid * _RPT, _RPT), :])

    return body(u_flat, src2d, dst2d)


def _dinv_of(deg_blk):
    return jax.lax.rsqrt(deg_blk[0, :, 0:1] + deg_blk[1, :, 0:1] + 1.0)


def _mm_scale_kernel(x_ref, w_ref, deg_ref, o_ref):
    t = jnp.dot(x_ref[...], w_ref[...], preferred_element_type=jnp.float32)
    u = t * _dinv_of(deg_ref)
    o_ref[0] = u[:, :_CH]
    o_ref[1] = u[:, _CH:]


def _mm_scale(x, w, deg16):
    return pl.pallas_call(
        _mm_scale_kernel,
        grid=(_N // _NB,),
        in_specs=[
            pl.BlockSpec((_NB, 256), lambda i: (i, 0)),
            pl.BlockSpec((256, 256), lambda i: (0, 0)),
            pl.BlockSpec((_NC, _NB, _CH), lambda i: (0, i, 0)),
        ],
        out_specs=pl.BlockSpec((_NC, _NB, _CH), lambda i: (0, i, 0)),
        out_shape=jax.ShapeDtypeStruct((_NC, _NP, _CH), jnp.float32),
    )(x, w, deg16)


def _mid_kernel(agg_ref, deg_ref, b1_ref, w_ref, o_ref):
    dinv = _dinv_of(deg_ref)
    h = jnp.concatenate([agg_ref[0], agg_ref[1]], axis=1)
    h = jnp.maximum(h * dinv + b1_ref[...], 0.0)
    t = jnp.dot(h, w_ref[...], preferred_element_type=jnp.float32)
    u = t * dinv
    o_ref[0] = u[:, :_CH]
    o_ref[1] = u[:, _CH:]


def _mid(agg1, deg16, b1, w2):
    return pl.pallas_call(
        _mid_kernel,
        grid=(_N // _NB,),
        in_specs=[
            pl.BlockSpec((_NC, _NB, _CH), lambda i: (0, i, 0)),
            pl.BlockSpec((_NC, _NB, _CH), lambda i: (0, i, 0)),
            pl.BlockSpec((1, 256), lambda i: (0, 0)),
            pl.BlockSpec((256, 256), lambda i: (0, 0)),
        ],
        out_specs=pl.BlockSpec((_NC, _NB, _CH), lambda i: (0, i, 0)),
        out_shape=jax.ShapeDtypeStruct((_NC, _NP, _CH), jnp.float32),
    )(agg1, deg16, b1, w2)


def _final_kernel(agg_ref, deg_ref, bmu_ref, bsig_ref, mu_ref, sig_ref):
    dinv = _dinv_of(deg_ref)
    mu_ref[...] = agg_ref[0] * dinv + bmu_ref[...]
    sig_ref[...] = agg_ref[1] * dinv + bsig_ref[...]


def _final(agg2, deg16, b_mu, b_sig):
    return pl.pallas_call(
        _final_kernel,
        grid=(_N // _NB,),
        in_specs=[
            pl.BlockSpec((_NC, _NB, _CH), lambda i: (0, i, 0)),
            pl.BlockSpec((_NC, _NB, _CH), lambda i: (0, i, 0)),
            pl.BlockSpec((1, _CH), lambda i: (0, 0)),
            pl.BlockSpec((1, _CH), lambda i: (0, 0)),
        ],
        out_specs=[
            pl.BlockSpec((_NB, _CH), lambda i: (i, 0)),
            pl.BlockSpec((_NB, _CH), lambda i: (i, 0)),
        ],
        out_shape=[
            jax.ShapeDtypeStruct((_N, _CH), jnp.float32),
            jax.ShapeDtypeStruct((_N, _CH), jnp.float32),
        ],
    )(agg2, deg16, b_mu, b_sig)


def kernel(x, edge_index, W1, b1, W_mu, b_mu, W_sig, b_sig):
    pad = _NCHUNK * _CHUNK - _E
    src = jnp.concatenate([edge_index[0].astype(jnp.int32),
                           jnp.zeros((pad,), jnp.int32)])
    dst = jnp.concatenate([edge_index[1].astype(jnp.int32),
                           jnp.full((pad,), _N, jnp.int32)])
    src2d = jnp.concatenate([src, src + _NP]).reshape(_NC * _NCHUNK, _CHUNK)
    dst2d = dst.reshape(_NCHUNK, _CHUNK)

    deg2 = _sc_degree(dst2d, jnp.ones((_CHUNK, _CH), jnp.float32),
                      jnp.zeros((_NP, _CH), jnp.float32))
    W2 = jnp.concatenate([W_mu, W_sig], axis=1)

    u1 = _sc_aggregate(_mm_scale(x, W1, deg2).reshape(_NC * _NP, _CH),
                       src2d, dst2d)
    u2 = _sc_aggregate(_mid(u1, deg2, b1.reshape(1, -1), W2)
                       .reshape(_NC * _NP, _CH), src2d, dst2d)
    return _final(u2, deg2, b_mu.reshape(1, _CH), b_sig.reshape(1, _CH))


# speedup vs baseline: 1.0002x; 1.0002x over previous
import functools

import jax
import jax.numpy as jnp
from jax import lax
from jax.experimental import pallas as pl
from jax.experimental.pallas import tpu as pltpu
from jax.experimental.pallas import tpu_sc as plsc

_N = 10000
_E = 160000
_NC = 2
_NS = 16
_CH = 128
_CHUNK = 128
_NCHUNK = 1280
_CPT = _NCHUNK // _NS
_HALF = _CPT // 2
_NP = 10240
_RPT = _NP // _NS
_NB = 1000


def _sc_mesh():
    return plsc.VectorSubcoreMesh(core_axis_name="c", subcore_axis_name="s")


_NW = _NC * _NS
_CPW = _NCHUNK // _NW


def _sc_degree(dst2d, ones_hbm, zeros_hbm):
    @functools.partial(
        pl.kernel,
        out_type=jax.ShapeDtypeStruct((_NC, _NP, _CH), jnp.float32),
        mesh=_sc_mesh(),
        scratch_types=[
            pltpu.VMEM((_CPW, _CHUNK), jnp.int32),
            pltpu.VMEM((_CHUNK, _CH), jnp.float32),
            pltpu.VMEM_SHARED((_NP, _CH), jnp.float32),
        ],
    )
    def body(dst_hbm, ones_hbm_ref, zeros_hbm_ref, out_hbm, dstm, onesv, acc):
        cid = lax.axis_index("c")
        sid = lax.axis_index("s")
        wid = cid * _NS + sid

        pltpu.sync_copy(ones_hbm_ref, onesv)
        pltpu.sync_copy(zeros_hbm_ref.at[pl.ds(sid * _RPT, _RPT), :],
                        acc.at[pl.ds(sid * _RPT, _RPT), :])
        pltpu.sync_copy(dst_hbm.at[pl.ds(wid * _CPW, _CPW), :], dstm)
        plsc.subcore_barrier()

        def step(k, c):
            pltpu.sync_copy(onesv, acc.at[dstm.at[k]], add=True)
            return c

        lax.fori_loop(0, _CPW, step, 0)

        plsc.subcore_barrier()
        pltpu.sync_copy(acc.at[pl.ds(sid * _RPT, _RPT), :],
                        out_hbm.at[cid, pl.ds(sid * _RPT, _RPT), :])

    return body(dst2d, ones_hbm, zeros_hbm)


def _sc_aggregate(u_flat, src2d, dst2d):
    @functools.partial(
        pl.kernel,
        out_type=jax.ShapeDtypeStruct((_NC, _NP, _CH), jnp.float32),
        mesh=_sc_mesh(),
        scratch_types=[
            pltpu.VMEM((_HALF, _CHUNK), jnp.int32),
            pltpu.VMEM((_HALF, _CHUNK), jnp.int32),
            pltpu.VMEM((_CHUNK, _CH), jnp.float32),
            pltpu.VMEM((_CHUNK, _CH), jnp.float32),
            pltpu.VMEM_SHARED((_NP, _CH), jnp.float32),
            pltpu.SemaphoreType.DMA,
            pltpu.SemaphoreType.DMA,
            pltpu.SemaphoreType.DMA,
            pltpu.SemaphoreType.DMA,
        ],
    )
    def body(u_hbm, src_hbm, dst_hbm, out_hbm, srcm, dstm, rows0, rows1,
             acc, gs0, gs1, ss0, ss1):
        cid = lax.axis_index("c")
        sid = lax.axis_index("s")
        pltpu.sync_copy(u_hbm.at[pl.ds(cid * _NP + sid * _RPT, _RPT), :],
                        acc.at[pl.ds(sid * _RPT, _RPT), :])
        plsc.subcore_barrier()

        def gather(k, buf, sem):
            pltpu.async_copy(u_hbm.at[srcm.at[k]], buf, sem)

        def gwait(buf, sem):
            pltpu.make_async_copy(u_hbm.at[srcm.at[0]], buf, sem).wait()

        def scatter(k, buf, sem):
            pltpu.async_copy(buf, acc.at[dstm.at[k]], sem, add=True)

        def swait(buf, sem):
            pltpu.make_async_copy(buf, acc.at[dstm.at[0]], sem).wait()

        def step(j, c):
            a = 2 * j

            @pl.when(j > 0)
            def _():
                swait(rows1, ss1)

            gather(a + 1, rows1, gs1)
            gwait(rows0, gs0)
            scatter(a, rows0, ss0)
            swait(rows0, ss0)
            gather(jnp.minimum(a + 2, _HALF - 1), rows0, gs0)
            gwait(rows1, gs1)
            scatter(a + 1, rows1, ss1)
            return c

        for h in range(_CPT // _HALF):
            cbase = sid * _CPT + h * _HALF
            pltpu.sync_copy(src_hbm.at[pl.ds(cid * _NCHUNK + cbase, _HALF), :],
                            srcm)
            pltpu.sync_copy(dst_hbm.at[pl.ds(cbase, _HALF), :], dstm)
            gather(0, rows0, gs0)
            lax.fori_loop(0, _HALF // 2, step, 0)
            gwait(rows0, gs0)
            swait(rows1, ss1)

        plsc.subcore_barrier()
        pltpu.sync_copy(acc.at[pl.ds(sid * _RPT, _RPT), :],
                        out_hbm.at[cid, pl.ds(sid * _RPT, _RPT), :])

    return body(u_flat, src2d, dst2d)


def _dinv_of(deg_blk):
    return jax.lax.rsqrt(deg_blk[0, :, 0:1] + deg_blk[1, :, 0:1] + 1.0)


def _mm_scale_kernel(x_ref, w_ref, deg_ref, o_ref):
    t = jnp.dot(x_ref[...], w_ref[...], preferred_element_type=jnp.float32)
    u = t * _dinv_of(deg_ref)
    o_ref[0] = u[:, :_CH]
    o_ref[1] = u[:, _CH:]


def _mm_scale(x, w, deg16):
    return pl.pallas_call(
        _mm_scale_kernel,
        grid=(_N // _NB,),
        in_specs=[
            pl.BlockSpec((_NB, 256), lambda i: (i, 0)),
            pl.BlockSpec((256, 256), lambda i: (0, 0)),
            pl.BlockSpec((_NC, _NB, _CH), lambda i: (0, i, 0)),
        ],
        out_specs=pl.BlockSpec((_NC, _NB, _CH), lambda i: (0, i, 0)),
        out_shape=jax.ShapeDtypeStruct((_NC, _NP, _CH), jnp.float32),
    )(x, w, deg16)


def _mid_kernel(agg_ref, deg_ref, b1_ref, w_ref, o_ref):
    dinv = _dinv_of(deg_ref)
    h = jnp.concatenate([agg_ref[0], agg_ref[1]], axis=1)
    h = jnp.maximum(h * dinv + b1_ref[...], 0.0)
    t = jnp.dot(h, w_ref[...], preferred_element_type=jnp.float32)
    u = t * dinv
    o_ref[0] = u[:, :_CH]
    o_ref[1] = u[:, _CH:]


def _mid(agg1, deg16, b1, w2):
    return pl.pallas_call(
        _mid_kernel,
        grid=(_N // _NB,),
        in_specs=[
            pl.BlockSpec((_NC, _NB, _CH), lambda i: (0, i, 0)),
            pl.BlockSpec((_NC, _NB, _CH), lambda i: (0, i, 0)),
            pl.BlockSpec((1, 256), lambda i: (0, 0)),
            pl.BlockSpec((256, 256), lambda i: (0, 0)),
        ],
        out_specs=pl.BlockSpec((_NC, _NB, _CH), lambda i: (0, i, 0)),
        out_shape=jax.ShapeDtypeStruct((_NC, _NP, _CH), jnp.float32),
    )(agg1, deg16, b1, w2)


def _final_kernel(agg_ref, deg_ref, bmu_ref, bsig_ref, mu_ref, sig_ref):
    dinv = _dinv_of(deg_ref)
    mu_ref[...] = agg_ref[0] * dinv + bmu_ref[...]
    sig_ref[...] = agg_ref[1] * dinv + bsig_ref[...]


def _final(agg2, deg16, b_mu, b_sig):
    return pl.pallas_call(
        _final_kernel,
        grid=(_N // _NB,),
        in_specs=[
            pl.BlockSpec((_NC, _NB, _CH), lambda i: (0, i, 0)),
            pl.BlockSpec((_NC, _NB, _CH), lambda i: (0, i, 0)),
            pl.BlockSpec((1, _CH), lambda i: (0, 0)),
            pl.BlockSpec((1, _CH), lambda i: (0, 0)),
        ],
        out_specs=[
            pl.BlockSpec((_NB, _CH), lambda i: (i, 0)),
            pl.BlockSpec((_NB, _CH), lambda i: (i, 0)),
        ],
        out_shape=[
            jax.ShapeDtypeStruct((_N, _CH), jnp.float32),
            jax.ShapeDtypeStruct((_N, _CH), jnp.float32),
        ],
    )(agg2, deg16, b_mu, b_sig)


def kernel(x, edge_index, W1, b1, W_mu, b_mu, W_sig, b_sig):
    pad = _NCHUNK * _CHUNK - _E
    src = jnp.concatenate([edge_index[0].astype(jnp.int32),
                           jnp.zeros((pad,), jnp.int32)])
    dst = jnp.concatenate([edge_index[1].astype(jnp.int32),
                           jnp.full((pad,), _N, jnp.int32)])
    src2d = jnp.concatenate([src, src + _NP]).reshape(_NC * _NCHUNK, _CHUNK)
    dst2d = dst.reshape(_NCHUNK, _CHUNK)

    deg2 = _sc_degree(dst2d, jnp.ones((_CHUNK, _CH), jnp.float32),
                      jnp.zeros((_NP, _CH), jnp.float32))
    W2 = jnp.concatenate([W_mu, W_sig], axis=1)

    u1 = _sc_aggregate(_mm_scale(x, W1, deg2).reshape(_NC * _NP, _CH),
                       src2d, dst2d)
    u2 = _sc_aggregate(_mid(u1, deg2, b1.reshape(1, -1), W2)
                       .reshape(_NC * _NP, _CH), src2d, dst2d)
    return _final(u2, deg2, b_mu.reshape(1, _CH), b_sig.reshape(1, _CH))
